# baseline (speedup 1.0000x reference)
import functools

import jax
import jax.numpy as jnp
from jax import lax
from jax.experimental import pallas as pl
from jax.experimental.pallas import tpu as pltpu
from jax.experimental.pallas import tpu_sc as plsc

B = 16384
D = 64
CDIM = 100

GCHUNK = 128


@functools.cache
def _build_sc_gather():
    info = plsc.get_sparse_core_info()
    nc, ns = info.num_cores, info.num_subcores
    nw = nc * ns
    b_per_w = B // nw
    nchunks = b_per_w // GCHUNK
    mesh = plsc.VectorSubcoreMesh(core_axis_name="c", subcore_axis_name="s")

    @functools.partial(
        pl.kernel,
        mesh=mesh,
        compiler_params=pltpu.CompilerParams(use_tc_tiling_on_sc=False),
        out_type=(
            jax.ShapeDtypeStruct((B, D), jnp.float32),
            jax.ShapeDtypeStruct((B, D), jnp.float32),
        ),
        scratch_types=[
            pltpu.VMEM((b_per_w,), jnp.int32),
            pltpu.VMEM((b_per_w,), jnp.int32),
            pltpu.VMEM((b_per_w, D), jnp.float32),
            pltpu.VMEM((b_per_w, D), jnp.float32),
            pltpu.SemaphoreType.DMA,
            pltpu.SemaphoreType.DMA,
        ],
    )
    def sc_gather(uid_hbm, iid_hbm, uemb_hbm, iemb_hbm, ue_out, ie_out,
                  uidx_v, iidx_v, urows_v, irows_v, usem, isem):
        wid = lax.axis_index("s") * nc + lax.axis_index("c")
        base = wid * b_per_w
        pltpu.sync_copy(uid_hbm.at[pl.ds(base, b_per_w)], uidx_v)
        pltpu.sync_copy(iid_hbm.at[pl.ds(base, b_per_w)], iidx_v)
        copies = []
        for j in range(nchunks):
            sl = pl.ds(j * GCHUNK, GCHUNK)
            copies.append(
                pltpu.async_copy(uemb_hbm.at[uidx_v.at[sl]], urows_v.at[sl], usem))
            copies.append(
                pltpu.async_copy(iemb_hbm.at[iidx_v.at[sl]], irows_v.at[sl], isem))
        for c in copies:
            c.wait()
        pltpu.sync_copy(urows_v, ue_out.at[pl.ds(base, b_per_w)])
        pltpu.sync_copy(irows_v, ie_out.at[pl.ds(base, b_per_w)])

    return sc_gather


BLK = 2048


def _tc_body(ue_ref, ie_ref, cf_ref, w1_ref, b1_ref, w2t_ref, b2_ref, out_ref):
    ue = ue_ref[...]
    ie = ie_ref[...]
    cf = cf_ref[...]
    mf = jnp.sum(ue * ie, axis=1, keepdims=True)
    w1 = w1_ref[...]
    h = (jnp.dot(ue, w1[:D, :], preferred_element_type=jnp.float32)
         + jnp.dot(cf, w1[D:, :], preferred_element_type=jnp.float32)
         + b1_ref[...])
    h = jnp.maximum(h, 0.0)
    mlp = jnp.sum(h * w2t_ref[...], axis=1, keepdims=True) + b2_ref[...]
    out_ref[...] = (mf + mlp) * 0.5


@functools.cache
def _build_tc_forward():
    grid = B // BLK
    return pl.pallas_call(
        _tc_body,
        grid=(grid,),
        in_specs=[
            pl.BlockSpec((BLK, D), lambda i: (i, 0)),
            pl.BlockSpec((BLK, D), lambda i: (i, 0)),
            pl.BlockSpec((BLK, CDIM), lambda i: (i, 0)),
            pl.BlockSpec((D + CDIM, D), lambda i: (0, 0)),
            pl.BlockSpec((1, D), lambda i: (0, 0)),
            pl.BlockSpec((1, D), lambda i: (0, 0)),
            pl.BlockSpec((1, 1), lambda i: (0, 0)),
        ],
        out_specs=pl.BlockSpec((BLK, 1), lambda i: (i, 0)),
        out_shape=jax.ShapeDtypeStruct((B, 1), jnp.float32),
    )


def kernel(user_ids, item_ids, content_features, user_emb, item_emb, W1, b1, W2, b2):
    ue, ie = _build_sc_gather()(user_ids, item_ids, user_emb, item_emb)
    return _build_tc_forward()(
        ue, ie, content_features, W1,
        b1.reshape(1, D), W2.reshape(1, D), b2.reshape(1, 1))
